# baseline (device time: 17264 ns/iter reference)
import os

import jax
import jax.numpy as jnp
from jax import lax
from jax.experimental import pallas as pl
from jax.experimental.pallas import tpu as pltpu

_MODE = os.environ.get("KMODE", "full")

N_DEV = 4
B = 2
SQ = 128
SKV_PER = 128
HQ = 4
DH = 64
WINDOW = 128
D_MODEL = 512
D_QK = 256
KV_ROWS = B * SKV_PER

SF = 0
SS = 1


def kernel(x, Wq, K_ext, V_ext, Wo):
    kv = jnp.concatenate(
        [K_ext.reshape(KV_ROWS, HQ * DH), V_ext.reshape(KV_ROWS, HQ * DH)],
        axis=0,
    ).astype(jnp.bfloat16)

    def body(x_ref, wq_ref, kv_ref, wo_ref, out_ref, gath_ref,
             send_sems, recv_sems):
        my_pos = lax.axis_index("i")
        left = lax.rem(my_pos + N_DEV - 1, N_DEV)
        right = lax.rem(my_pos + 1, N_DEV)

        if _MODE != "nocomm":
            barrier_sem = pltpu.get_barrier_semaphore()
            pl.semaphore_signal(barrier_sem, inc=1, device_id=(left,),
                                device_id_type=pl.DeviceIdType.MESH)
            pl.semaphore_signal(barrier_sem, inc=1, device_id=(right,),
                                device_id_type=pl.DeviceIdType.MESH)
            pl.semaphore_wait(barrier_sem, 2)

        def start_send(src_ref, dst_slot, send_idx, dev):
            pltpu.make_async_remote_copy(
                src_ref=src_ref,
                dst_ref=gath_ref.at[dst_slot],
                send_sem=send_sems.at[send_idx],
                recv_sem=recv_sems.at[dst_slot],
                device_id=(dev,),
                device_id_type=pl.DeviceIdType.MESH,
            ).start()

        def wait_recv(slot):
            pltpu.make_async_remote_copy(
                src_ref=gath_ref.at[slot], dst_ref=gath_ref.at[slot],
                send_sem=send_sems.at[0], recv_sem=recv_sems.at[slot],
                device_id=(my_pos,), device_id_type=pl.DeviceIdType.MESH,
            ).wait_recv()

        def wait_send(send_idx):
            pltpu.make_async_remote_copy(
                src_ref=gath_ref.at[SF], dst_ref=gath_ref.at[SF],
                send_sem=send_sems.at[send_idx], recv_sem=recv_sems.at[SF],
                device_id=(my_pos,), device_id_type=pl.DeviceIdType.MESH,
            ).wait_send()

        if _MODE == "nocomm":
            gath_ref[SF] = kv_ref[...]
            gath_ref[SS] = kv_ref[...]
        else:
            @pl.when(my_pos == 0)
            def _():
                start_send(kv_ref, SF, 1, 3)
                start_send(kv_ref, SS, 0, 1)
                gath_ref[SF] = kv_ref[...]

            @pl.when(my_pos == 1)
            def _():
                start_send(kv_ref, SF, 1, 2)
                start_send(kv_ref, SS, 0, 0)
                gath_ref[SF] = kv_ref[...]

        wq_b16 = wq_ref[...].astype(jnp.bfloat16)
        q_proj = [
            jnp.dot(x_ref[b].astype(jnp.bfloat16), wq_b16,
                    preferred_element_type=jnp.float32
                    ).astype(jnp.bfloat16)
            for b in range(B)
        ]

        if _MODE != "nocomm":
            @pl.when(my_pos == 2)
            def _():
                wait_recv(SF)
                start_send(gath_ref.at[SF], SS, 0, 3)
            @pl.when(my_pos == 3)
            def _():
                wait_recv(SF)
                start_send(gath_ref.at[SF], SS, 0, 2)

        f_is_1 = (my_pos == 1) | (my_pos == 2)
        f_off = jnp.where(f_is_1, SKV_PER, 0)
        s_off = jnp.where(f_is_1, 0, SKV_PER)

        qi = lax.broadcasted_iota(jnp.int32, (SQ, SKV_PER), 0)
        kj = lax.broadcasted_iota(jnp.int32, (SQ, SKV_PER), 1)

        def block_scores(slot, off, b, h):
            r0 = b * SKV_PER
            c0 = h * DH
            k_blk = gath_ref[slot, r0:r0 + SKV_PER, c0:c0 + DH]
            v_blk = gath_ref[slot,
                             KV_ROWS + r0:KV_ROWS + r0 + SKV_PER,
                             c0:c0 + DH]
            q_bh = q_proj[b][:, c0:c0 + DH]
            s = lax.dot_general(
                q_bh, k_blk, (((1,), (1,)), ((), ())),
                preferred_element_type=jnp.float32) * 0.125
            s = jnp.where(jnp.abs(qi - (kj + off)) <= WINDOW, s, -1e9)
            return s, v_blk

        def pv(p, v_blk):
            return jnp.dot(p.astype(jnp.bfloat16), v_blk,
                           preferred_element_type=jnp.float32)

        if _MODE != "nocompute":
            parts = []
            for b in range(B):
                for h in range(HQ):
                    s_f, v_f = block_scores(SF, f_off, b, h)
                    m_f = jnp.max(s_f, axis=1, keepdims=True)
                    p_f = jnp.exp(s_f - m_f)
                    l_f = jnp.sum(p_f, axis=1, keepdims=True)
                    parts.append((m_f, l_f, pv(p_f, v_f)))

        if _MODE != "nocomm":
            wait_recv(SS)

        if _MODE == "nocompute":
            out_ref[...] = jnp.zeros((B, SQ, D_MODEL), jnp.float32)
        else:
            wo_b16 = wo_ref[...].astype(jnp.bfloat16)
            for b in range(B):
                ctx_heads = []
                for h in range(HQ):
                    m_f, l_f, c_f = parts[b * HQ + h]
                    s_s, v_s = block_scores(SS, s_off, b, h)
                    m = jnp.maximum(m_f,
                                    jnp.max(s_s, axis=1, keepdims=True))
                    alpha = jnp.exp(m_f - m)
                    p_s = jnp.exp(s_s - m)
                    c = c_f * alpha + pv(p_s, v_s)
                    l = l_f * alpha + jnp.sum(p_s, axis=1, keepdims=True)
                    ctx_heads.append(c / l)
                ctx_b = jnp.concatenate(ctx_heads, axis=1)
                out_ref[b] = jnp.dot(ctx_b.astype(jnp.bfloat16), wo_b16,
                                     preferred_element_type=jnp.float32)

        if _MODE != "nocomm":
            @pl.when((my_pos == 0) | (my_pos == 1))
            def _():
                wait_send(0)
                wait_send(1)

            @pl.when((my_pos == 2) | (my_pos == 3))
            def _():
                wait_send(0)

    return pl.pallas_call(
        body,
        out_shape=jax.ShapeDtypeStruct((B, SQ, D_MODEL), jnp.float32),
        in_specs=[pl.BlockSpec(memory_space=pltpu.VMEM)] * 4,
        out_specs=pl.BlockSpec(memory_space=pltpu.VMEM),
        scratch_shapes=[
            pltpu.VMEM((2, 2 * KV_ROWS, HQ * DH), jnp.bfloat16),
            pltpu.SemaphoreType.DMA((2,)),
            pltpu.SemaphoreType.DMA((2,)),
        ],
        compiler_params=pltpu.CompilerParams(collective_id=0),
    )(x, Wq, kv, Wo)
